# Initial kernel scaffold; baseline (speedup 1.0000x reference)
import functools

import jax
import jax.numpy as jnp
from jax import lax
from jax.experimental import pallas as pl
from jax.experimental.pallas import tpu as pltpu
from jax.experimental.pallas import tpu_sc as plsc

_NC = 2
_NS = 16
_NW = _NC * _NS
_SUB = 128
_K = 8
_CHUNK = _SUB * _K


@functools.lru_cache(maxsize=None)
def _make_gather(n, vocab, dim):
    per_w = n // _NW
    nrow = per_w // _SUB
    nchunk = per_w // _CHUNK

    mesh = plsc.VectorSubcoreMesh(core_axis_name="c", subcore_axis_name="s")

    @functools.partial(
        pl.kernel,
        mesh=mesh,
        out_type=jax.ShapeDtypeStruct((n, dim), jnp.float32),
        scratch_types=[
            pltpu.VMEM((nrow, _SUB), jnp.int32),
            pltpu.VMEM((_CHUNK, dim), jnp.float32),
            pltpu.SemaphoreType.DMA,
        ],
    )
    def gather_kernel(table_hbm, idx_hbm, out_hbm, idx_v, rows_v, sem):
        wid = lax.axis_index("s") * _NC + lax.axis_index("c")
        base = wid * per_w
        pltpu.sync_copy(idx_hbm.at[pl.ds(wid * nrow, nrow)], idx_v)

        def chunk_body(g, carry):
            handles = [
                pltpu.async_copy(
                    table_hbm.at[idx_v.at[g * _K + j]],
                    rows_v.at[pl.ds(j * _SUB, _SUB)],
                    sem,
                )
                for j in range(_K)
            ]
            for h in handles:
                h.wait()
            pltpu.sync_copy(rows_v, out_hbm.at[pl.ds(base + g * _CHUNK, _CHUNK)])
            return carry

        lax.fori_loop(0, nchunk, chunk_body, 0)

    return gather_kernel


def kernel(x, table):
    b, l = x.shape
    vocab, dim = table.shape
    n = b * l
    idx = x.reshape(n // _SUB, _SUB).astype(jnp.int32)
    out = _make_gather(n, vocab, dim)(table, idx)
    return out.reshape(b, l, dim)


# speedup vs baseline: 1.4779x; 1.4779x over previous
import functools

import jax
import jax.numpy as jnp
from jax import lax
from jax.experimental import pallas as pl
from jax.experimental.pallas import tpu as pltpu
from jax.experimental.pallas import tpu_sc as plsc

_NC = 2
_NS = 16
_NW = _NC * _NS
_SUB = 128
_K = 8
_CHUNK = _SUB * _K


@functools.lru_cache(maxsize=None)
def _make_gather(n, vocab, dim):
    per_w = n // _NW
    nrow = per_w // _SUB
    nchunk = per_w // _CHUNK

    mesh = plsc.VectorSubcoreMesh(core_axis_name="c", subcore_axis_name="s")

    @functools.partial(
        pl.kernel,
        mesh=mesh,
        out_type=jax.ShapeDtypeStruct((n, dim), jnp.float32),
        scratch_types=[
            pltpu.VMEM((nrow, _SUB), jnp.int32),
            pltpu.VMEM((_CHUNK, dim), jnp.float32),
            pltpu.SemaphoreType.DMA,
        ],
        compiler_params=pltpu.CompilerParams(use_tc_tiling_on_sc=False),
    )
    def gather_kernel(table_hbm, idx_hbm, out_hbm, idx_v, rows_v, sem):
        wid = lax.axis_index("s") * _NC + lax.axis_index("c")
        base = wid * per_w
        pltpu.sync_copy(idx_hbm.at[pl.ds(wid * nrow, nrow)], idx_v)

        def chunk_body(g, carry):
            handles = [
                pltpu.async_copy(
                    table_hbm.at[idx_v.at[g * _K + j]],
                    rows_v.at[pl.ds(j * _SUB, _SUB)],
                    sem,
                )
                for j in range(_K)
            ]
            for h in handles:
                h.wait()
            pltpu.sync_copy(rows_v, out_hbm.at[pl.ds(base + g * _CHUNK, _CHUNK)])
            return carry

        lax.fori_loop(0, nchunk, chunk_body, 0)

    return gather_kernel


def kernel(x, table):
    b, l = x.shape
    vocab, dim = table.shape
    n = b * l
    idx = x.reshape(n // _SUB, _SUB).astype(jnp.int32)
    out = _make_gather(n, vocab, dim)(table, idx)
    return out.reshape(b, l, dim)


# speedup vs baseline: 1.4847x; 1.0046x over previous
import functools

import jax
import jax.numpy as jnp
from jax import lax
from jax.experimental import pallas as pl
from jax.experimental.pallas import tpu as pltpu
from jax.experimental.pallas import tpu_sc as plsc

_NC = 2
_NS = 16
_NW = _NC * _NS
_SUB = 128
_K = 4
_CHUNK = _SUB * _K


@functools.lru_cache(maxsize=None)
def _make_gather(n, vocab, dim):
    per_w = n // _NW
    nrow = per_w // _SUB
    nchunk = per_w // _CHUNK
    niter = nchunk // 2

    mesh = plsc.VectorSubcoreMesh(core_axis_name="c", subcore_axis_name="s")

    @functools.partial(
        pl.kernel,
        mesh=mesh,
        out_type=jax.ShapeDtypeStruct((n, dim), jnp.float32),
        scratch_types=[
            pltpu.VMEM((nrow, _SUB), jnp.int32),
            pltpu.VMEM((_CHUNK, dim), jnp.float32),
            pltpu.VMEM((_CHUNK, dim), jnp.float32),
            pltpu.SemaphoreType.DMA,
            pltpu.SemaphoreType.DMA,
            pltpu.SemaphoreType.DMA,
            pltpu.SemaphoreType.DMA,
        ],
        compiler_params=pltpu.CompilerParams(use_tc_tiling_on_sc=False),
    )
    def gather_kernel(table_hbm, idx_hbm, out_hbm, idx_v, rows0, rows1,
                      gsem0, gsem1, wsem0, wsem1):
        wid = lax.axis_index("s") * _NC + lax.axis_index("c")
        base = wid * per_w
        pltpu.sync_copy(idx_hbm.at[pl.ds(wid * nrow, nrow)], idx_v)

        def fire(g, buf, sem):
            return [
                pltpu.async_copy(
                    table_hbm.at[idx_v.at[g * _K + j]],
                    buf.at[pl.ds(j * _SUB, _SUB)],
                    sem,
                )
                for j in range(_K)
            ]

        def body(i, carry):
            g0 = 2 * i
            @pl.when(i > 0)
            def _():
                pltpu.make_async_copy(
                    rows0, out_hbm.at[pl.ds(base, _CHUNK)], wsem0).wait()
                pltpu.make_async_copy(
                    rows1, out_hbm.at[pl.ds(base, _CHUNK)], wsem1).wait()
            h0 = fire(g0, rows0, gsem0)
            h1 = fire(g0 + 1, rows1, gsem1)
            for h in h0:
                h.wait()
            pltpu.async_copy(
                rows0, out_hbm.at[pl.ds(base + g0 * _CHUNK, _CHUNK)], wsem0)
            for h in h1:
                h.wait()
            pltpu.async_copy(
                rows1, out_hbm.at[pl.ds(base + (g0 + 1) * _CHUNK, _CHUNK)], wsem1)
            return carry

        lax.fori_loop(0, niter, body, 0)
        pltpu.make_async_copy(rows0, out_hbm.at[pl.ds(base, _CHUNK)], wsem0).wait()
        pltpu.make_async_copy(rows1, out_hbm.at[pl.ds(base, _CHUNK)], wsem1).wait()

    return gather_kernel


def kernel(x, table):
    b, l = x.shape
    vocab, dim = table.shape
    n = b * l
    idx = x.reshape(n // _SUB, _SUB).astype(jnp.int32)
    out = _make_gather(n, vocab, dim)(table, idx)
    return out.reshape(b, l, dim)
